# baseline (speedup 1.0000x reference)
import functools

import jax
import jax.numpy as jnp
from jax import lax
from jax.experimental import pallas as pl
from jax.experimental.pallas import tpu as pltpu
from jax.experimental.pallas import tpu_sc as plsc

T = 2048
D = 768
FF = 2048
E = 64
K = 2
A = T * K
M = 128
NT = 96
TM = 256

NW = 32
GR = NT * M // NW
GC = 128
NCH = GR // GC
CT = T // NW

_SQRT1_2 = 0.7071067811865476


def _router_body(x_ref, wr_ref, idx_ref, w_ref):
    x = x_ref[...]
    logits = jax.lax.dot_general(
        x, wr_ref[...], (((1,), (0,)), ((), ())),
        preferred_element_type=jnp.float32)
    m = jnp.max(logits, axis=-1, keepdims=True)
    p = jnp.exp(logits - m)
    p = p / jnp.sum(p, axis=-1, keepdims=True)
    lane = jax.lax.broadcasted_iota(jnp.int32, p.shape, 1)
    v1 = jnp.max(p, axis=-1, keepdims=True)
    i1 = jnp.min(jnp.where(p >= v1, lane, E), axis=-1, keepdims=True)
    p2 = jnp.where(lane == i1, -jnp.inf, p)
    v2 = jnp.max(p2, axis=-1, keepdims=True)
    i2 = jnp.min(jnp.where(p2 >= v2, lane, E), axis=-1, keepdims=True)
    s = v1 + v2
    idx_ref[...] = jnp.concatenate([i1, i2], axis=1)
    w_ref[...] = jnp.concatenate([v1 / s, v2 / s], axis=1)


def _route(x, W_router):
    return pl.pallas_call(
        _router_body,
        grid=(T // TM,),
        in_specs=[
            pl.BlockSpec((TM, D), lambda t: (t, 0)),
            pl.BlockSpec((D, E), lambda t: (0, 0)),
        ],
        out_specs=[
            pl.BlockSpec((TM, K), lambda t: (t, 0)),
            pl.BlockSpec((TM, K), lambda t: (t, 0)),
        ],
        out_shape=[
            jax.ShapeDtypeStruct((T, K), jnp.int32),
            jax.ShapeDtypeStruct((T, K), jnp.float32),
        ],
    )(x, W_router)


def _plan(idx, w):
    ids = idx.reshape(A)
    wf = w.reshape(A)
    order = jnp.argsort(ids, stable=True).astype(jnp.int32)
    sorted_e = ids[order]
    sorted_tok = (order // K).astype(jnp.int32)
    sorted_w = wf[order]
    counts = jnp.bincount(ids, length=E).astype(jnp.int32)
    tiles_pe = (counts + M - 1) // M
    tile_cum = jnp.cumsum(tiles_pe)
    tile_start = tile_cum - tiles_pe
    tt = jnp.arange(NT, dtype=jnp.int32)
    te_raw = jnp.clip(
        jnp.searchsorted(tile_cum, tt, side="right"), 0, E - 1).astype(jnp.int32)
    real = tt < tile_cum[-1]
    nvalid = jnp.where(
        real, jnp.clip(counts[te_raw] - (tt - tile_start[te_raw]) * M, 0, M),
        0).astype(jnp.int32)
    last_e = jnp.max(ids).astype(jnp.int32)
    tile_expert = jnp.where(real, te_raw, last_e).astype(jnp.int32)
    cnt_excl = jnp.cumsum(counts) - counts
    rloc = jnp.arange(A, dtype=jnp.int32) - cnt_excl[sorted_e]
    dest = (tile_start[sorted_e] * M + rloc).astype(jnp.int32)
    padded_tok = jnp.zeros(NT * M, jnp.int32).at[dest].set(sorted_tok)
    padded_w = jnp.zeros((NT * M, 1), jnp.float32).at[dest, 0].set(sorted_w)
    inv = jnp.zeros(A, jnp.int32).at[order].set(dest)
    slots = inv.reshape(T, K)
    return tile_expert, padded_tok, nvalid, padded_w, slots[:, 0], slots[:, 1]


def _sc_gather_body(x_hbm, tok_hbm, xg_hbm, idx_v, rows_v, sem):
    wid = lax.axis_index("s") * 2 + lax.axis_index("c")
    base = wid * GR
    for c in range(NCH):
        off = base + c * GC
        pltpu.sync_copy(tok_hbm.at[pl.ds(off, GC)], idx_v)
        pltpu.async_copy(x_hbm.at[idx_v], rows_v, sem).wait()
        pltpu.sync_copy(rows_v, xg_hbm.at[pl.ds(off, GC)])


_sc_gather = functools.partial(
    pl.kernel,
    out_type=jax.ShapeDtypeStruct((NT * M, D), jnp.float32),
    mesh=plsc.VectorSubcoreMesh(core_axis_name="c", subcore_axis_name="s"),
    scratch_types=[
        pltpu.VMEM((GC,), jnp.int32),
        pltpu.VMEM((GC, D), jnp.float32),
        pltpu.SemaphoreType.DMA,
    ],
)(_sc_gather_body)


def _moe_body(te_ref, nv_ref, xg_ref, w1_ref, v_ref, w2_ref, wc_ref, os_ref):
    nv = nv_ref[pl.program_id(0)]

    @pl.when(nv > 0)
    def _compute():
        xv = xg_ref[...]
        h = jnp.dot(xv, w1_ref[0], preferred_element_type=jnp.float32)
        g = h * 0.5 * (1.0 + jax.lax.erf(h * _SQRT1_2))
        hv = jnp.dot(xv, v_ref[0], preferred_element_type=jnp.float32)
        o = jnp.dot(g * hv, w2_ref[0], preferred_element_type=jnp.float32)
        os_ref[...] = o * wc_ref[...]


def _w_map(t, te, nv):
    return te[t], 0, 0


def _os_map(t, te, nv):
    return jnp.where(nv[t] == 0, NT - 1, t), 0


def _moe(xg, W1, V, W2, tile_expert, nvalid, padded_w):
    grid_spec = pltpu.PrefetchScalarGridSpec(
        num_scalar_prefetch=2,
        grid=(NT,),
        in_specs=[
            pl.BlockSpec((M, D), lambda t, te, nv: (t, 0)),
            pl.BlockSpec((1, D, FF), _w_map),
            pl.BlockSpec((1, D, FF), _w_map),
            pl.BlockSpec((1, FF, D), _w_map),
            pl.BlockSpec((M, 1), lambda t, te, nv: (t, 0)),
        ],
        out_specs=pl.BlockSpec((M, D), _os_map),
    )
    return pl.pallas_call(
        _moe_body,
        grid_spec=grid_spec,
        out_shape=jax.ShapeDtypeStruct((NT * M, D), jnp.float32),
    )(tile_expert, nvalid, xg, W1, V, W2, padded_w)


def _sc_combine_body(os_hbm, s0_hbm, s1_hbm, out_hbm, i0, i1, r0, r1, sem):
    wid = lax.axis_index("s") * 2 + lax.axis_index("c")
    base = wid * CT
    pltpu.sync_copy(s0_hbm.at[pl.ds(base, CT)], i0)
    pltpu.sync_copy(s1_hbm.at[pl.ds(base, CT)], i1)
    pltpu.async_copy(os_hbm.at[i0], r0, sem).wait()
    pltpu.async_copy(os_hbm.at[i1], r1, sem).wait()

    def row(i, c):
        for j in range(D // 16):
            sl = pl.ds(j * 16, 16)
            r0[i, sl] = r0[i, sl] + r1[i, sl]
        return c

    lax.fori_loop(0, CT, row, 0)
    pltpu.sync_copy(r0, out_hbm.at[pl.ds(base, CT)])


_sc_combine = functools.partial(
    pl.kernel,
    out_type=jax.ShapeDtypeStruct((T, D), jnp.float32),
    mesh=plsc.VectorSubcoreMesh(core_axis_name="c", subcore_axis_name="s"),
    scratch_types=[
        pltpu.VMEM((CT,), jnp.int32),
        pltpu.VMEM((CT,), jnp.int32),
        pltpu.VMEM((CT, D), jnp.float32),
        pltpu.VMEM((CT, D), jnp.float32),
        pltpu.SemaphoreType.DMA,
    ],
)(_sc_combine_body)


def kernel(hidden_states, W_router, W1, V, W2):
    b, s, d = hidden_states.shape
    x = hidden_states.reshape(-1, d)
    idx, w = _route(x, W_router)
    tile_expert, padded_tok, nvalid, padded_w, slot0, slot1 = _plan(idx, w)
    xg = _sc_gather(x, padded_tok)
    osorted = _moe(xg, W1, V, W2, tile_expert, nvalid, padded_w)
    out = _sc_combine(osorted, slot0, slot1)
    return out.reshape(b, s, d)


# speedup vs baseline: 1.5461x; 1.5461x over previous
import functools

import jax
import jax.numpy as jnp
from jax import lax
from jax.experimental import pallas as pl
from jax.experimental.pallas import tpu as pltpu
from jax.experimental.pallas import tpu_sc as plsc

T = 2048
D = 768
FF = 2048
E = 64
K = 2
A = T * K
M = 128
NT = 96
TM = 256

NW = 32
GR = NT * M // NW
GC = 128
NCH = GR // GC
CT = T // NW

_SQRT1_2 = 0.7071067811865476


def _router_body(x_ref, wr_ref, idx_ref, w_ref):
    x = x_ref[...]
    logits = jax.lax.dot_general(
        x, wr_ref[...], (((1,), (0,)), ((), ())),
        preferred_element_type=jnp.float32)
    m = jnp.max(logits, axis=-1, keepdims=True)
    p = jnp.exp(logits - m)
    p = p / jnp.sum(p, axis=-1, keepdims=True)
    lane = jax.lax.broadcasted_iota(jnp.int32, p.shape, 1)
    v1 = jnp.max(p, axis=-1, keepdims=True)
    i1 = jnp.min(jnp.where(p >= v1, lane, E), axis=-1, keepdims=True)
    p2 = jnp.where(lane == i1, -jnp.inf, p)
    v2 = jnp.max(p2, axis=-1, keepdims=True)
    i2 = jnp.min(jnp.where(p2 >= v2, lane, E), axis=-1, keepdims=True)
    s = v1 + v2
    idx_ref[...] = jnp.concatenate([i1, i2], axis=1)
    w_ref[...] = jnp.concatenate([v1 / s, v2 / s], axis=1)


def _route(x, W_router):
    return pl.pallas_call(
        _router_body,
        grid=(T // TM,),
        in_specs=[
            pl.BlockSpec((TM, D), lambda t: (t, 0)),
            pl.BlockSpec((D, E), lambda t: (0, 0)),
        ],
        out_specs=[
            pl.BlockSpec((TM, K), lambda t: (t, 0)),
            pl.BlockSpec((TM, K), lambda t: (t, 0)),
        ],
        out_shape=[
            jax.ShapeDtypeStruct((T, K), jnp.int32),
            jax.ShapeDtypeStruct((T, K), jnp.float32),
        ],
    )(x, W_router)


def _plan(idx, w):
    ids = idx.reshape(A)
    wf = w.reshape(A)
    order = jnp.argsort(ids, stable=True).astype(jnp.int32)
    sorted_e = ids[order]
    sorted_tok = (order // K).astype(jnp.int32)
    sorted_w = wf[order]
    counts = jnp.bincount(ids, length=E).astype(jnp.int32)
    tiles_pe = (counts + M - 1) // M
    tile_cum = jnp.cumsum(tiles_pe)
    tile_start = tile_cum - tiles_pe
    tt = jnp.arange(NT, dtype=jnp.int32)
    te_raw = jnp.clip(
        jnp.searchsorted(tile_cum, tt, side="right"), 0, E - 1).astype(jnp.int32)
    real = tt < tile_cum[-1]
    nvalid = jnp.where(
        real, jnp.clip(counts[te_raw] - (tt - tile_start[te_raw]) * M, 0, M),
        0).astype(jnp.int32)
    last_e = jnp.max(ids).astype(jnp.int32)
    tile_expert = jnp.where(real, te_raw, last_e).astype(jnp.int32)
    cnt_excl = jnp.cumsum(counts) - counts
    rloc = jnp.arange(A, dtype=jnp.int32) - cnt_excl[sorted_e]
    dest = (tile_start[sorted_e] * M + rloc).astype(jnp.int32)
    pad_fill = jnp.arange(NT * M, dtype=jnp.int32) % T
    padded_tok = pad_fill.at[dest].set(sorted_tok)
    padded_w = jnp.zeros((NT * M, 1), jnp.float32).at[dest, 0].set(sorted_w)
    inv = jnp.zeros(A, jnp.int32).at[order].set(dest)
    slots = inv.reshape(T, K)
    return tile_expert, padded_tok, nvalid, padded_w, slots[:, 0], slots[:, 1]


def _sc_gather_body(x_hbm, tok_hbm, xg_hbm, idx_v, rows_v, sem):
    wid = lax.axis_index("s") * 2 + lax.axis_index("c")
    base = wid * GR
    for c in range(NCH):
        off = base + c * GC
        pltpu.sync_copy(tok_hbm.at[pl.ds(off, GC)], idx_v)
        pltpu.async_copy(x_hbm.at[idx_v], rows_v, sem).wait()
        pltpu.sync_copy(rows_v, xg_hbm.at[pl.ds(off, GC)])


_sc_gather = functools.partial(
    pl.kernel,
    out_type=jax.ShapeDtypeStruct((NT * M, D), jnp.float32),
    mesh=plsc.VectorSubcoreMesh(core_axis_name="c", subcore_axis_name="s"),
    scratch_types=[
        pltpu.VMEM((GC,), jnp.int32),
        pltpu.VMEM((GC, D), jnp.float32),
        pltpu.SemaphoreType.DMA,
    ],
)(_sc_gather_body)


def _moe_body(te_ref, nv_ref, xg_ref, w1_ref, v_ref, w2_ref, wc_ref, os_ref):
    nv = nv_ref[pl.program_id(0)]

    @pl.when(nv > 0)
    def _compute():
        xv = xg_ref[...]
        h = jnp.dot(xv, w1_ref[0], preferred_element_type=jnp.float32)
        g = h * 0.5 * (1.0 + jax.lax.erf(h * _SQRT1_2))
        hv = jnp.dot(xv, v_ref[0], preferred_element_type=jnp.float32)
        o = jnp.dot(g * hv, w2_ref[0], preferred_element_type=jnp.float32)
        os_ref[...] = o * wc_ref[...]


def _w_map(t, te, nv):
    return te[t], 0, 0


def _os_map(t, te, nv):
    return jnp.where(nv[t] == 0, NT - 1, t), 0


def _moe(xg, W1, V, W2, tile_expert, nvalid, padded_w):
    grid_spec = pltpu.PrefetchScalarGridSpec(
        num_scalar_prefetch=2,
        grid=(NT,),
        in_specs=[
            pl.BlockSpec((M, D), lambda t, te, nv: (t, 0)),
            pl.BlockSpec((1, D, FF), _w_map),
            pl.BlockSpec((1, D, FF), _w_map),
            pl.BlockSpec((1, FF, D), _w_map),
            pl.BlockSpec((M, 1), lambda t, te, nv: (t, 0)),
        ],
        out_specs=pl.BlockSpec((M, D), _os_map),
    )
    return pl.pallas_call(
        _moe_body,
        grid_spec=grid_spec,
        out_shape=jax.ShapeDtypeStruct((NT * M, D), jnp.float32),
    )(tile_expert, nvalid, xg, W1, V, W2, padded_w)


def _sc_combine_body(os_hbm, s0_hbm, s1_hbm, out_hbm, i0, i1, r0, r1, sem):
    wid = lax.axis_index("s") * 2 + lax.axis_index("c")
    base = wid * CT
    pltpu.sync_copy(s0_hbm.at[pl.ds(base, CT)], i0)
    pltpu.sync_copy(s1_hbm.at[pl.ds(base, CT)], i1)
    pltpu.async_copy(os_hbm.at[i0], r0, sem).wait()
    pltpu.async_copy(os_hbm.at[i1], r1, sem).wait()

    def row(i, c):
        for j in range(D // 16):
            sl = pl.ds(j * 16, 16)
            r0[i, sl] = r0[i, sl] + r1[i, sl]
        return c

    lax.fori_loop(0, CT, row, 0)
    pltpu.sync_copy(r0, out_hbm.at[pl.ds(base, CT)])


_sc_combine = functools.partial(
    pl.kernel,
    out_type=jax.ShapeDtypeStruct((T, D), jnp.float32),
    mesh=plsc.VectorSubcoreMesh(core_axis_name="c", subcore_axis_name="s"),
    scratch_types=[
        pltpu.VMEM((CT,), jnp.int32),
        pltpu.VMEM((CT,), jnp.int32),
        pltpu.VMEM((CT, D), jnp.float32),
        pltpu.VMEM((CT, D), jnp.float32),
        pltpu.SemaphoreType.DMA,
    ],
)(_sc_combine_body)


def kernel(hidden_states, W_router, W1, V, W2):
    b, s, d = hidden_states.shape
    x = hidden_states.reshape(-1, d)
    idx, w = _route(x, W_router)
    tile_expert, padded_tok, nvalid, padded_w, slot0, slot1 = _plan(idx, w)
    xg = _sc_gather(x, padded_tok)
    osorted = _moe(xg, W1, V, W2, tile_expert, nvalid, padded_w)
    out = _sc_combine(osorted, slot0, slot1)
    return out.reshape(b, s, d)


# speedup vs baseline: 1.8398x; 1.1899x over previous
import functools

import jax
import jax.numpy as jnp
from jax import lax
from jax.experimental import pallas as pl
from jax.experimental.pallas import tpu as pltpu
from jax.experimental.pallas import tpu_sc as plsc

T = 2048
D = 768
FF = 2048
E = 64
K = 2
A = T * K
M = 128
NT = 96
TM = 256
NBLK = T // TM

NW = 32
CT = T // NW

_SQRT1_2 = 0.7071067811865476


def _router_body(x_ref, wr_ref, idx_ref, w_ref, rank_ref, cnt_ref, carry):
    t = pl.program_id(0)

    @pl.when(t == 0)
    def _init():
        carry[...] = jnp.zeros_like(carry)

    x = x_ref[...]
    logits = jax.lax.dot_general(
        x, wr_ref[...], (((1,), (0,)), ((), ())),
        preferred_element_type=jnp.float32)
    m = jnp.max(logits, axis=-1, keepdims=True)
    p = jnp.exp(logits - m)
    p = p / jnp.sum(p, axis=-1, keepdims=True)
    lane = jax.lax.broadcasted_iota(jnp.int32, p.shape, 1)
    v1 = jnp.max(p, axis=-1, keepdims=True)
    i1 = jnp.min(jnp.where(p >= v1, lane, E), axis=-1, keepdims=True)
    p2 = jnp.where(lane == i1, -jnp.inf, p)
    v2 = jnp.max(p2, axis=-1, keepdims=True)
    i2 = jnp.min(jnp.where(p2 >= v2, lane, E), axis=-1, keepdims=True)
    s = v1 + v2
    idx_ref[...] = jnp.concatenate([i1, i2], axis=1)
    w_ref[...] = jnp.concatenate([v1 / s, v2 / s], axis=1)

    oh1 = (lane == i1).astype(jnp.float32)
    oh2 = (lane == i2).astype(jnp.float32)
    oh = oh1 + oh2
    r_i = jax.lax.broadcasted_iota(jnp.int32, (TM, TM), 0)
    c_i = jax.lax.broadcasted_iota(jnp.int32, (TM, TM), 1)
    ltri = (c_i < r_i).astype(jnp.float32)
    cum_excl = jnp.dot(ltri, oh, preferred_element_type=jnp.float32)
    base = cum_excl + carry[0:1, :]
    rank1 = jnp.sum(base * oh1, axis=1, keepdims=True)
    rank2 = jnp.sum(base * oh2, axis=1, keepdims=True)
    rank_ref[...] = jnp.concatenate([rank1, rank2], axis=1).astype(jnp.int32)
    carry[0:1, :] = carry[0:1, :] + jnp.sum(oh, axis=0, keepdims=True)

    @pl.when(t == NBLK - 1)
    def _emit_counts():
        cnt_ref[...] = carry[0:1, :].astype(jnp.int32)


def _route(x, W_router):
    return pl.pallas_call(
        _router_body,
        grid=(NBLK,),
        in_specs=[
            pl.BlockSpec((TM, D), lambda t: (t, 0)),
            pl.BlockSpec((D, E), lambda t: (0, 0)),
        ],
        out_specs=[
            pl.BlockSpec((TM, K), lambda t: (t, 0)),
            pl.BlockSpec((TM, K), lambda t: (t, 0)),
            pl.BlockSpec((TM, K), lambda t: (t, 0)),
            pl.BlockSpec((1, E), lambda t: (0, 0)),
        ],
        out_shape=[
            jax.ShapeDtypeStruct((T, K), jnp.int32),
            jax.ShapeDtypeStruct((T, K), jnp.float32),
            jax.ShapeDtypeStruct((T, K), jnp.int32),
            jax.ShapeDtypeStruct((1, E), jnp.int32),
        ],
        scratch_shapes=[pltpu.VMEM((8, E), jnp.float32)],
    )(x, W_router)


def _plan(idx, w, rank, cnt):
    ids = idx.reshape(A)
    wf = w.reshape(A)
    counts = cnt.reshape(E)
    tiles_pe = (counts + M - 1) // M
    tile_cum = jnp.cumsum(tiles_pe)
    tile_start = tile_cum - tiles_pe
    dest = (tile_start[ids] * M + rank.reshape(A)).astype(jnp.int32)
    tt = jnp.arange(NT, dtype=jnp.int32)
    te_raw = jnp.sum(
        (tt[:, None] >= tile_cum[None, :]).astype(jnp.int32), axis=1)
    te_raw = jnp.clip(te_raw, 0, E - 1)
    real = tt < tile_cum[-1]
    nvalid = jnp.where(
        real, jnp.clip(counts[te_raw] - (tt - tile_start[te_raw]) * M, 0, M),
        0).astype(jnp.int32)
    last_e = jnp.max(jnp.where(counts > 0, jnp.arange(E, dtype=jnp.int32), -1))
    tile_expert = jnp.where(real, te_raw, last_e).astype(jnp.int32)
    padded_w = jnp.zeros((NT * M, 1), jnp.float32).at[dest, 0].set(wf)
    slots = dest.reshape(T, K)
    return tile_expert, nvalid, padded_w, slots[:, 0], slots[:, 1]


def _sc_dispatch_body(x_hbm, d0_hbm, d1_hbm, xg_hbm, i0, i1, rows, sem):
    wid = lax.axis_index("s") * 2 + lax.axis_index("c")
    base = wid * CT
    pltpu.sync_copy(d0_hbm.at[pl.ds(base, CT)], i0)
    pltpu.sync_copy(d1_hbm.at[pl.ds(base, CT)], i1)
    pltpu.sync_copy(x_hbm.at[pl.ds(base, CT)], rows)
    pltpu.async_copy(rows, xg_hbm.at[i0], sem).wait()
    pltpu.async_copy(rows, xg_hbm.at[i1], sem).wait()


_sc_dispatch = functools.partial(
    pl.kernel,
    out_type=jax.ShapeDtypeStruct((NT * M, D), jnp.float32),
    mesh=plsc.VectorSubcoreMesh(core_axis_name="c", subcore_axis_name="s"),
    scratch_types=[
        pltpu.VMEM((CT,), jnp.int32),
        pltpu.VMEM((CT,), jnp.int32),
        pltpu.VMEM((CT, D), jnp.float32),
        pltpu.SemaphoreType.DMA,
    ],
)(_sc_dispatch_body)


def _moe_body(te_ref, nv_ref, xg_ref, w1_ref, v_ref, w2_ref, wc_ref, os_ref):
    nv = nv_ref[pl.program_id(0)]

    @pl.when(nv > 0)
    def _compute():
        xv = xg_ref[...]
        h = jnp.dot(xv, w1_ref[0], preferred_element_type=jnp.float32)
        g = h * 0.5 * (1.0 + jax.lax.erf(h * _SQRT1_2))
        hv = jnp.dot(xv, v_ref[0], preferred_element_type=jnp.float32)
        o = jnp.dot(g * hv, w2_ref[0], preferred_element_type=jnp.float32)
        os_ref[...] = o * wc_ref[...]


def _w_map(t, te, nv):
    return te[t], 0, 0


def _os_map(t, te, nv):
    return jnp.where(nv[t] == 0, NT - 1, t), 0


def _moe(xg, W1, V, W2, tile_expert, nvalid, padded_w):
    grid_spec = pltpu.PrefetchScalarGridSpec(
        num_scalar_prefetch=2,
        grid=(NT,),
        in_specs=[
            pl.BlockSpec((M, D), lambda t, te, nv: (t, 0)),
            pl.BlockSpec((1, D, FF), _w_map),
            pl.BlockSpec((1, D, FF), _w_map),
            pl.BlockSpec((1, FF, D), _w_map),
            pl.BlockSpec((M, 1), lambda t, te, nv: (t, 0)),
        ],
        out_specs=pl.BlockSpec((M, D), _os_map),
    )
    return pl.pallas_call(
        _moe_body,
        grid_spec=grid_spec,
        out_shape=jax.ShapeDtypeStruct((NT * M, D), jnp.float32),
    )(tile_expert, nvalid, xg, W1, V, W2, padded_w)


def _sc_combine_body(os_hbm, s0_hbm, s1_hbm, out_hbm, i0, i1, r0, r1, sem):
    wid = lax.axis_index("s") * 2 + lax.axis_index("c")
    base = wid * CT
    pltpu.sync_copy(s0_hbm.at[pl.ds(base, CT)], i0)
    pltpu.sync_copy(s1_hbm.at[pl.ds(base, CT)], i1)
    pltpu.async_copy(os_hbm.at[i0], r0, sem).wait()
    pltpu.async_copy(os_hbm.at[i1], r1, sem).wait()

    def row(i, c):
        for j in range(D // 16):
            sl = pl.ds(j * 16, 16)
            r0[i, sl] = r0[i, sl] + r1[i, sl]
        return c

    lax.fori_loop(0, CT, row, 0)
    pltpu.sync_copy(r0, out_hbm.at[pl.ds(base, CT)])


_sc_combine = functools.partial(
    pl.kernel,
    out_type=jax.ShapeDtypeStruct((T, D), jnp.float32),
    mesh=plsc.VectorSubcoreMesh(core_axis_name="c", subcore_axis_name="s"),
    scratch_types=[
        pltpu.VMEM((CT,), jnp.int32),
        pltpu.VMEM((CT,), jnp.int32),
        pltpu.VMEM((CT, D), jnp.float32),
        pltpu.VMEM((CT, D), jnp.float32),
        pltpu.SemaphoreType.DMA,
    ],
)(_sc_combine_body)


def kernel(hidden_states, W_router, W1, V, W2):
    b, s, d = hidden_states.shape
    x = hidden_states.reshape(-1, d)
    idx, w, rank, cnt = _route(x, W_router)
    tile_expert, nvalid, padded_w, slot0, slot1 = _plan(idx, w, rank, cnt)
    xg = _sc_dispatch(x, slot0, slot1)
    osorted = _moe(xg, W1, V, W2, tile_expert, nvalid, padded_w)
    out = _sc_combine(osorted, slot0, slot1)
    return out.reshape(b, s, d)


# speedup vs baseline: 2.0751x; 1.1279x over previous
import functools

import jax
import jax.numpy as jnp
from jax import lax
from jax.experimental import pallas as pl
from jax.experimental.pallas import tpu as pltpu
from jax.experimental.pallas import tpu_sc as plsc

T = 2048
D = 768
FF = 2048
E = 64
K = 2
A = T * K
M = 128
NT = 96
TM = 256
NBLK = T // TM

NW = 32
CT = T // NW

_SQRT1_2 = 0.7071067811865476


def _router_body(x_ref, wr_ref, idx_ref, w_ref, rank_ref, cnt_ref, carry):
    t = pl.program_id(0)

    @pl.when(t == 0)
    def _init():
        carry[...] = jnp.zeros_like(carry)

    x = x_ref[...]
    logits = jax.lax.dot_general(
        x, wr_ref[...], (((1,), (0,)), ((), ())),
        preferred_element_type=jnp.float32)
    m = jnp.max(logits, axis=-1, keepdims=True)
    p = jnp.exp(logits - m)
    p = p / jnp.sum(p, axis=-1, keepdims=True)
    lane = jax.lax.broadcasted_iota(jnp.int32, p.shape, 1)
    v1 = jnp.max(p, axis=-1, keepdims=True)
    i1 = jnp.min(jnp.where(p >= v1, lane, E), axis=-1, keepdims=True)
    p2 = jnp.where(lane == i1, -jnp.inf, p)
    v2 = jnp.max(p2, axis=-1, keepdims=True)
    i2 = jnp.min(jnp.where(p2 >= v2, lane, E), axis=-1, keepdims=True)
    s = v1 + v2
    idx_ref[...] = jnp.concatenate([i1, i2], axis=1)
    w_ref[...] = jnp.concatenate([v1 / s, v2 / s], axis=1)

    oh1 = (lane == i1).astype(jnp.float32)
    oh2 = (lane == i2).astype(jnp.float32)
    oh = oh1 + oh2
    r_i = jax.lax.broadcasted_iota(jnp.int32, (TM, TM), 0)
    c_i = jax.lax.broadcasted_iota(jnp.int32, (TM, TM), 1)
    ltri = (c_i < r_i).astype(jnp.float32)
    cum_excl = jnp.dot(ltri, oh, preferred_element_type=jnp.float32)
    base = cum_excl + carry[0:1, :]
    rank1 = jnp.sum(base * oh1, axis=1, keepdims=True)
    rank2 = jnp.sum(base * oh2, axis=1, keepdims=True)
    rank_ref[...] = jnp.concatenate([rank1, rank2], axis=1).astype(jnp.int32)
    carry[0:1, :] = carry[0:1, :] + jnp.sum(oh, axis=0, keepdims=True)

    @pl.when(t == NBLK - 1)
    def _emit_counts():
        cnt_ref[...] = carry[0:1, :].astype(jnp.int32)


def _route(x, W_router):
    return pl.pallas_call(
        _router_body,
        grid=(NBLK,),
        in_specs=[
            pl.BlockSpec((TM, D), lambda t: (t, 0)),
            pl.BlockSpec((D, E), lambda t: (0, 0)),
        ],
        out_specs=[
            pl.BlockSpec((TM, K), lambda t: (t, 0)),
            pl.BlockSpec((TM, K), lambda t: (t, 0)),
            pl.BlockSpec((TM, K), lambda t: (t, 0)),
            pl.BlockSpec((1, E), lambda t: (0, 0)),
        ],
        out_shape=[
            jax.ShapeDtypeStruct((T, K), jnp.int32),
            jax.ShapeDtypeStruct((T, K), jnp.float32),
            jax.ShapeDtypeStruct((T, K), jnp.int32),
            jax.ShapeDtypeStruct((1, E), jnp.int32),
        ],
        scratch_shapes=[pltpu.VMEM((8, E), jnp.float32)],
    )(x, W_router)


def _plan(idx, w, rank, cnt):
    ids = idx.reshape(A)
    wf = w.reshape(A)
    counts = cnt.reshape(E)
    tiles_pe = (counts + M - 1) // M
    tile_cum = jnp.cumsum(tiles_pe)
    tile_start = tile_cum - tiles_pe
    ts_f = tile_start.astype(jnp.float32)
    cn_f = counts.astype(jnp.float32)
    oh_a = (ids[:, None] == jnp.arange(E, dtype=jnp.int32)[None, :]
            ).astype(jnp.float32)
    dest = ((oh_a @ ts_f).astype(jnp.int32) * M + rank.reshape(A)
            ).astype(jnp.int32)
    tt = jnp.arange(NT, dtype=jnp.int32)
    te_raw = jnp.sum(
        (tt[:, None] >= tile_cum[None, :]).astype(jnp.int32), axis=1)
    te_raw = jnp.clip(te_raw, 0, E - 1)
    oh_t = (te_raw[:, None] == jnp.arange(E, dtype=jnp.int32)[None, :]
            ).astype(jnp.float32)
    cnt_t = (oh_t @ cn_f).astype(jnp.int32)
    start_t = (oh_t @ ts_f).astype(jnp.int32)
    real = tt < tile_cum[-1]
    nvalid = jnp.where(
        real, jnp.clip(cnt_t - (tt - start_t) * M, 0, M), 0).astype(jnp.int32)
    last_e = jnp.max(jnp.where(counts > 0, jnp.arange(E, dtype=jnp.int32), -1))
    tile_expert = jnp.where(real, te_raw, last_e).astype(jnp.int32)
    padded_w = jnp.zeros((NT * M, 1), jnp.float32).at[dest, 0].set(wf)
    slots = dest.reshape(T, K)
    return tile_expert, nvalid, padded_w, slots[:, 0], slots[:, 1]


def _sc_dispatch_body(x_hbm, d0_hbm, d1_hbm, xg_hbm, i0, i1, rows, sem):
    wid = lax.axis_index("s") * 2 + lax.axis_index("c")
    base = wid * CT
    pltpu.sync_copy(d0_hbm.at[pl.ds(base, CT)], i0)
    pltpu.sync_copy(d1_hbm.at[pl.ds(base, CT)], i1)
    pltpu.sync_copy(x_hbm.at[pl.ds(base, CT)], rows)
    pltpu.async_copy(rows, xg_hbm.at[i0], sem).wait()
    pltpu.async_copy(rows, xg_hbm.at[i1], sem).wait()


_sc_dispatch = functools.partial(
    pl.kernel,
    out_type=jax.ShapeDtypeStruct((NT * M, D), jnp.float32),
    mesh=plsc.VectorSubcoreMesh(core_axis_name="c", subcore_axis_name="s"),
    scratch_types=[
        pltpu.VMEM((CT,), jnp.int32),
        pltpu.VMEM((CT,), jnp.int32),
        pltpu.VMEM((CT, D), jnp.float32),
        pltpu.SemaphoreType.DMA,
    ],
)(_sc_dispatch_body)


def _moe_body(te_ref, nv_ref, xg_ref, w1_ref, v_ref, w2_ref, wc_ref, os_ref):
    nv = nv_ref[pl.program_id(0)]

    @pl.when(nv > 0)
    def _compute():
        xv = xg_ref[...]
        h = jnp.dot(xv, w1_ref[0], preferred_element_type=jnp.float32)
        g = h * 0.5 * (1.0 + jax.lax.erf(h * _SQRT1_2))
        hv = jnp.dot(xv, v_ref[0], preferred_element_type=jnp.float32)
        o = jnp.dot(g * hv, w2_ref[0], preferred_element_type=jnp.float32)
        os_ref[...] = o * wc_ref[...]


def _w_map(t, te, nv):
    return te[t], 0, 0


def _os_map(t, te, nv):
    return jnp.where(nv[t] == 0, NT - 1, t), 0


def _xg_map(t, te, nv):
    return jnp.where(nv[t] == 0, NT - 1, t), 0


def _moe(xg, W1, V, W2, tile_expert, nvalid, padded_w):
    grid_spec = pltpu.PrefetchScalarGridSpec(
        num_scalar_prefetch=2,
        grid=(NT,),
        in_specs=[
            pl.BlockSpec((M, D), _xg_map),
            pl.BlockSpec((1, D, FF), _w_map),
            pl.BlockSpec((1, D, FF), _w_map),
            pl.BlockSpec((1, FF, D), _w_map),
            pl.BlockSpec((M, 1), _xg_map),
        ],
        out_specs=pl.BlockSpec((M, D), _os_map),
    )
    return pl.pallas_call(
        _moe_body,
        grid_spec=grid_spec,
        out_shape=jax.ShapeDtypeStruct((NT * M, D), jnp.float32),
    )(tile_expert, nvalid, xg, W1, V, W2, padded_w)


def _sc_combine_body(os_hbm, s0_hbm, s1_hbm, out_hbm, i0, i1, r0, r1, sem):
    wid = lax.axis_index("s") * 2 + lax.axis_index("c")
    base = wid * CT
    pltpu.sync_copy(s0_hbm.at[pl.ds(base, CT)], i0)
    pltpu.sync_copy(s1_hbm.at[pl.ds(base, CT)], i1)
    pltpu.async_copy(os_hbm.at[i0], r0, sem).wait()
    pltpu.async_copy(os_hbm.at[i1], r1, sem).wait()

    def row(i, c):
        for j in range(D // 16):
            sl = pl.ds(j * 16, 16)
            r0[i, sl] = r0[i, sl] + r1[i, sl]
        return c

    lax.fori_loop(0, CT, row, 0)
    pltpu.sync_copy(r0, out_hbm.at[pl.ds(base, CT)])


_sc_combine = functools.partial(
    pl.kernel,
    out_type=jax.ShapeDtypeStruct((T, D), jnp.float32),
    mesh=plsc.VectorSubcoreMesh(core_axis_name="c", subcore_axis_name="s"),
    scratch_types=[
        pltpu.VMEM((CT,), jnp.int32),
        pltpu.VMEM((CT,), jnp.int32),
        pltpu.VMEM((CT, D), jnp.float32),
        pltpu.VMEM((CT, D), jnp.float32),
        pltpu.SemaphoreType.DMA,
    ],
)(_sc_combine_body)


def kernel(hidden_states, W_router, W1, V, W2):
    b, s, d = hidden_states.shape
    x = hidden_states.reshape(-1, d)
    idx, w, rank, cnt = _route(x, W_router)
    tile_expert, nvalid, padded_w, slot0, slot1 = _plan(idx, w, rank, cnt)
    xg = _sc_dispatch(x, slot0, slot1)
    osorted = _moe(xg, W1, V, W2, tile_expert, nvalid, padded_w)
    out = _sc_combine(osorted, slot0, slot1)
    return out.reshape(b, s, d)


# speedup vs baseline: 2.0803x; 1.0025x over previous
import functools

import jax
import jax.numpy as jnp
from jax import lax
from jax.experimental import pallas as pl
from jax.experimental.pallas import tpu as pltpu
from jax.experimental.pallas import tpu_sc as plsc

T = 2048
D = 768
FF = 2048
E = 64
K = 2
A = T * K
M = 128
NT = 96
TM = 256
NBLK = T // TM

NW = 32
CT = T // NW

_SQRT1_2 = 0.7071067811865476


def _router_body(x_ref, wr_ref, idx_ref, w_ref, rank_ref, cnt_ref, carry):
    t = pl.program_id(0)

    @pl.when(t == 0)
    def _init():
        carry[...] = jnp.zeros_like(carry)

    x = x_ref[...]
    logits = jax.lax.dot_general(
        x, wr_ref[...], (((1,), (0,)), ((), ())),
        preferred_element_type=jnp.float32)
    m = jnp.max(logits, axis=-1, keepdims=True)
    p = jnp.exp(logits - m)
    p = p / jnp.sum(p, axis=-1, keepdims=True)
    lane = jax.lax.broadcasted_iota(jnp.int32, p.shape, 1)
    v1 = jnp.max(p, axis=-1, keepdims=True)
    i1 = jnp.min(jnp.where(p >= v1, lane, E), axis=-1, keepdims=True)
    p2 = jnp.where(lane == i1, -jnp.inf, p)
    v2 = jnp.max(p2, axis=-1, keepdims=True)
    i2 = jnp.min(jnp.where(p2 >= v2, lane, E), axis=-1, keepdims=True)
    s = v1 + v2
    idx_ref[...] = jnp.concatenate([i1, i2], axis=1)
    w_ref[...] = jnp.concatenate([v1 / s, v2 / s], axis=1)

    oh1 = (lane == i1).astype(jnp.float32)
    oh2 = (lane == i2).astype(jnp.float32)
    oh = oh1 + oh2
    r_i = jax.lax.broadcasted_iota(jnp.int32, (TM, TM), 0)
    c_i = jax.lax.broadcasted_iota(jnp.int32, (TM, TM), 1)
    ltri = (c_i < r_i).astype(jnp.float32)
    cum_excl = jnp.dot(ltri, oh, preferred_element_type=jnp.float32)
    base = cum_excl + carry[0:1, :]
    rank1 = jnp.sum(base * oh1, axis=1, keepdims=True)
    rank2 = jnp.sum(base * oh2, axis=1, keepdims=True)
    rank_ref[...] = jnp.concatenate([rank1, rank2], axis=1).astype(jnp.int32)
    carry[0:1, :] = carry[0:1, :] + jnp.sum(oh, axis=0, keepdims=True)

    @pl.when(t == NBLK - 1)
    def _emit_counts():
        cnt_ref[...] = carry[0:1, :].astype(jnp.int32)


def _route(x, W_router):
    return pl.pallas_call(
        _router_body,
        grid=(NBLK,),
        in_specs=[
            pl.BlockSpec((TM, D), lambda t: (t, 0)),
            pl.BlockSpec((D, E), lambda t: (0, 0)),
        ],
        out_specs=[
            pl.BlockSpec((TM, K), lambda t: (t, 0)),
            pl.BlockSpec((TM, K), lambda t: (t, 0)),
            pl.BlockSpec((TM, K), lambda t: (t, 0)),
            pl.BlockSpec((1, E), lambda t: (0, 0)),
        ],
        out_shape=[
            jax.ShapeDtypeStruct((T, K), jnp.int32),
            jax.ShapeDtypeStruct((T, K), jnp.float32),
            jax.ShapeDtypeStruct((T, K), jnp.int32),
            jax.ShapeDtypeStruct((1, E), jnp.int32),
        ],
        scratch_shapes=[pltpu.VMEM((8, E), jnp.float32)],
    )(x, W_router)


def _plan(idx, w, rank, cnt):
    ids = idx.reshape(A)
    wf = w.reshape(A)
    counts = cnt.reshape(E)
    tiles_pe = (counts + M - 1) // M
    tile_cum = jnp.cumsum(tiles_pe)
    tile_start = tile_cum - tiles_pe
    ts_f = tile_start.astype(jnp.float32)
    cn_f = counts.astype(jnp.float32)
    oh_a = (ids[:, None] == jnp.arange(E, dtype=jnp.int32)[None, :]
            ).astype(jnp.float32)
    dest = ((oh_a @ ts_f).astype(jnp.int32) * M + rank.reshape(A)
            ).astype(jnp.int32)
    tt = jnp.arange(NT, dtype=jnp.int32)
    te_raw = jnp.sum(
        (tt[:, None] >= tile_cum[None, :]).astype(jnp.int32), axis=1)
    te_raw = jnp.clip(te_raw, 0, E - 1)
    oh_t = (te_raw[:, None] == jnp.arange(E, dtype=jnp.int32)[None, :]
            ).astype(jnp.float32)
    cnt_t = (oh_t @ cn_f).astype(jnp.int32)
    start_t = (oh_t @ ts_f).astype(jnp.int32)
    real = tt < tile_cum[-1]
    nvalid = jnp.where(
        real, jnp.clip(cnt_t - (tt - start_t) * M, 0, M), 0).astype(jnp.int32)
    last_e = jnp.max(jnp.where(counts > 0, jnp.arange(E, dtype=jnp.int32), -1))
    tile_expert = jnp.where(real, te_raw, last_e).astype(jnp.int32)
    padded_w = jnp.zeros((NT * M, 1), jnp.float32).at[dest, 0].set(wf)
    slots = dest.reshape(T, K)
    return tile_expert, nvalid, padded_w, slots[:, 0], slots[:, 1]


def _sc_dispatch_body(x_hbm, d0_hbm, d1_hbm, xg_hbm, i0, i1, rows, sem):
    wid = lax.axis_index("s") * 2 + lax.axis_index("c")
    base = wid * CT
    c0 = pltpu.async_copy(d0_hbm.at[pl.ds(base, CT)], i0, sem)
    c1 = pltpu.async_copy(d1_hbm.at[pl.ds(base, CT)], i1, sem)
    c2 = pltpu.async_copy(x_hbm.at[pl.ds(base, CT)], rows, sem)
    c0.wait()
    c1.wait()
    c2.wait()
    s0 = pltpu.async_copy(rows, xg_hbm.at[i0], sem)
    s1 = pltpu.async_copy(rows, xg_hbm.at[i1], sem)
    s0.wait()
    s1.wait()


_sc_dispatch = functools.partial(
    pl.kernel,
    out_type=jax.ShapeDtypeStruct((NT * M, D), jnp.float32),
    mesh=plsc.VectorSubcoreMesh(core_axis_name="c", subcore_axis_name="s"),
    scratch_types=[
        pltpu.VMEM((CT,), jnp.int32),
        pltpu.VMEM((CT,), jnp.int32),
        pltpu.VMEM((CT, D), jnp.float32),
        pltpu.SemaphoreType.DMA,
    ],
)(_sc_dispatch_body)


def _moe_body(te_ref, nv_ref, xg_ref, w1_ref, v_ref, w2_ref, wc_ref, os_ref):
    nv = nv_ref[pl.program_id(0)]

    @pl.when(nv > 0)
    def _compute():
        xv = xg_ref[...]
        h = jnp.dot(xv, w1_ref[0], preferred_element_type=jnp.float32)
        g = h * 0.5 * (1.0 + jax.lax.erf(h * _SQRT1_2))
        hv = jnp.dot(xv, v_ref[0], preferred_element_type=jnp.float32)
        o = jnp.dot(g * hv, w2_ref[0], preferred_element_type=jnp.float32)
        os_ref[...] = o * wc_ref[...]


def _w_map(t, te, nv):
    return te[t], 0, 0


def _os_map(t, te, nv):
    return jnp.where(nv[t] == 0, NT - 1, t), 0


def _xg_map(t, te, nv):
    return jnp.where(nv[t] == 0, NT - 1, t), 0


def _moe(xg, W1, V, W2, tile_expert, nvalid, padded_w):
    grid_spec = pltpu.PrefetchScalarGridSpec(
        num_scalar_prefetch=2,
        grid=(NT,),
        in_specs=[
            pl.BlockSpec((M, D), _xg_map),
            pl.BlockSpec((1, D, FF), _w_map),
            pl.BlockSpec((1, D, FF), _w_map),
            pl.BlockSpec((1, FF, D), _w_map),
            pl.BlockSpec((M, 1), _xg_map),
        ],
        out_specs=pl.BlockSpec((M, D), _os_map),
    )
    return pl.pallas_call(
        _moe_body,
        grid_spec=grid_spec,
        out_shape=jax.ShapeDtypeStruct((NT * M, D), jnp.float32),
    )(tile_expert, nvalid, xg, W1, V, W2, padded_w)


def _sc_combine_body(os_hbm, s0_hbm, s1_hbm, out_hbm, i0, i1, r0, r1, sem):
    wid = lax.axis_index("s") * 2 + lax.axis_index("c")
    base = wid * CT
    c0 = pltpu.async_copy(s0_hbm.at[pl.ds(base, CT)], i0, sem)
    c1 = pltpu.async_copy(s1_hbm.at[pl.ds(base, CT)], i1, sem)
    c0.wait()
    c1.wait()
    g0 = pltpu.async_copy(os_hbm.at[i0], r0, sem)
    g1 = pltpu.async_copy(os_hbm.at[i1], r1, sem)
    g0.wait()
    g1.wait()

    def row(i, c):
        for j in range(D // 16):
            sl = pl.ds(j * 16, 16)
            r0[i, sl] = r0[i, sl] + r1[i, sl]
        return c

    lax.fori_loop(0, CT, row, 0)
    pltpu.sync_copy(r0, out_hbm.at[pl.ds(base, CT)])


_sc_combine = functools.partial(
    pl.kernel,
    out_type=jax.ShapeDtypeStruct((T, D), jnp.float32),
    mesh=plsc.VectorSubcoreMesh(core_axis_name="c", subcore_axis_name="s"),
    scratch_types=[
        pltpu.VMEM((CT,), jnp.int32),
        pltpu.VMEM((CT,), jnp.int32),
        pltpu.VMEM((CT, D), jnp.float32),
        pltpu.VMEM((CT, D), jnp.float32),
        pltpu.SemaphoreType.DMA,
    ],
)(_sc_combine_body)


def kernel(hidden_states, W_router, W1, V, W2):
    b, s, d = hidden_states.shape
    x = hidden_states.reshape(-1, d)
    idx, w, rank, cnt = _route(x, W_router)
    tile_expert, nvalid, padded_w, slot0, slot1 = _plan(idx, w, rank, cnt)
    xg = _sc_dispatch(x, slot0, slot1)
    osorted = _moe(xg, W1, V, W2, tile_expert, nvalid, padded_w)
    out = _sc_combine(osorted, slot0, slot1)
    return out.reshape(b, s, d)
